# Initial kernel scaffold; baseline (speedup 1.0000x reference)
import jax
import jax.numpy as jnp
from jax.experimental import pallas as pl


def kernel(net_output, target_structure, bboxes):
    raise NotImplementedError("write your pallas kernel here")


# speedup vs baseline: 29.4608x; 29.4608x over previous
import functools

import jax
import jax.numpy as jnp
from jax import lax
from jax.experimental import pallas as pl
from jax.experimental.pallas import tpu as pltpu
from jax.experimental.pallas import tpu_sc as plsc

_INTERPRET = False

B, C, S = 2, 8, 96
SPATIAL = S * S * S
ROWS = B * C
N_TOP = int(round(SPATIAL * 0.1))
ZSLAB = 8
NZ = S // ZSLAB

NTILES = 32
HALF = SPATIAL // 2
NBUCK = 2048
HB = NBUCK * 16
CHUNK = 16384
NCHUNK = HALF // CHUNK


def _loss_body(lo_ref, hi_ref, net_ref, tgt_ref, out_ref):
    b = pl.program_id(0)
    s = pl.program_id(1)
    z0 = s * ZSLAB
    zc = z0 + lax.broadcasted_iota(jnp.int32, (ZSLAB, S, S), 0)
    yc = lax.broadcasted_iota(jnp.int32, (ZSLAB, S, S), 1)
    xc = lax.broadcasted_iota(jnp.int32, (ZSLAB, S, S), 2)
    jc2 = lax.broadcasted_iota(jnp.int32, (S, S), 0)
    kc2 = lax.broadcasted_iota(jnp.int32, (S, S), 1)
    dmax = jnp.zeros((ZSLAB, S, S), jnp.float32)

    def bce(x, d):
        return jnp.maximum(x, 0.0) - x * d + jnp.log1p(jnp.exp(-jnp.abs(x)))

    for c in range(C - 1):
        lo0 = lo_ref[b, c, 0]
        lo1 = lo_ref[b, c, 1]
        lo2 = lo_ref[b, c, 2]
        hi0 = hi_ref[b, c, 0]
        hi1 = hi_ref[b, c, 1]
        hi2 = hi_ref[b, c, 2]
        rows = []
        for z in range(ZSLAB):
            zz = jnp.clip(z0 + z - lo0, 0, S - 1)
            rows.append(tgt_ref[0, pl.ds(zz, 1), :, :])
        g = jnp.concatenate(rows, axis=0)
        py = ((jc2 - lo1 + S) % S == kc2).astype(jnp.float32)
        px = ((jc2 - lo2 + S) % S == kc2).astype(jnp.float32)
        g = jax.lax.dot_general(py, g, (((1,), (1,)), ((), ())),
                                preferred_element_type=jnp.float32)
        g = jax.lax.dot_general(g, px, (((2,), (1,)), ((), ())),
                                preferred_element_type=jnp.float32)
        g = jnp.swapaxes(g, 0, 1)
        inside = ((zc >= lo0) & (zc < hi0)
                  & (yc >= lo1) & (yc < hi1)
                  & (xc >= lo2) & (xc < hi2))
        d = jnp.where(inside, g, 0.0)
        dmax = jnp.maximum(dmax, d)
        x = net_ref[0, c]
        out_ref[0, c] = bce(x, d)
    x = net_ref[0, C - 1]
    out_ref[0, C - 1] = bce(x, dmax)


def _tc_loss(net, tgt, lo, hi):
    return pl.pallas_call(
        _loss_body,
        grid=(B, NZ),
        in_specs=[
            pl.BlockSpec(memory_space=pltpu.SMEM),
            pl.BlockSpec(memory_space=pltpu.SMEM),
            pl.BlockSpec((1, C, ZSLAB, S, S), lambda b, s: (b, 0, s, 0, 0)),
            pl.BlockSpec((1, S, S, S), lambda b, s: (b, 0, 0, 0)),
        ],
        out_specs=pl.BlockSpec((1, C, ZSLAB, S, S), lambda b, s: (b, 0, s, 0, 0)),
        out_shape=jax.ShapeDtypeStruct((B, C, S, S, S), jnp.float32),
        interpret=_INTERPRET,
    )(lo, hi, net, tgt)


@functools.lru_cache(maxsize=None)
def _make_sc_hist(shift):
    mesh = plsc.VectorSubcoreMesh(core_axis_name="c", subcore_axis_name="s")

    @functools.partial(
        pl.kernel,
        mesh=mesh,
        out_type=[
            jax.ShapeDtypeStruct((NTILES, HB), jnp.float32),
            jax.ShapeDtypeStruct((NTILES, HB), jnp.float32),
        ],
        scratch_types=[
            pltpu.VMEM((CHUNK,), jnp.float32),
            pltpu.VMEM((HB,), jnp.float32),
            pltpu.VMEM((HB,), jnp.float32),
            pltpu.VMEM((16,), jnp.int32),
        ],
        compiler_params=pltpu.CompilerParams(needs_layout_passes=False),
    )
    def sc_hist(loss_hbm, base_hbm, cnt_hbm, sum_hbm, buf, hcnt, hsum, bvec):
        wid = lax.axis_index("s") * 2 + lax.axis_index("c")
        start = wid * HALF
        zeros = jnp.zeros((16,), jnp.float32)

        def zbody(i, _):
            hcnt[pl.ds(i * 16, 16)] = zeros
            hsum[pl.ds(i * 16, 16)] = zeros
            return 0

        lax.fori_loop(0, HB // 16, zbody, 0)

        pltpu.sync_copy(base_hbm.at[wid], bvec)
        base = bvec[...]
        lane = lax.iota(jnp.int32, 16)
        ones = jnp.full((16,), 1.0, jnp.float32)

        def chunk_body(ci, _):
            pltpu.sync_copy(loss_hbm.at[pl.ds(start + ci * CHUNK, CHUNK)], buf)

            def vbody(i, _):
                v = buf[pl.ds(i * 16, 16)]
                k = lax.bitcast_convert_type(v, jnp.int32)
                d = (k - base) >> shift
                m = (d >= 0) & (d < NBUCK)
                idx = jnp.clip(d, 0, NBUCK - 1) * 16 + lane
                plsc.addupdate_scatter(hcnt, [idx], ones, mask=m)
                plsc.addupdate_scatter(hsum, [idx], v, mask=m)
                return 0

            lax.fori_loop(0, CHUNK // 16, vbody, 0)
            return 0

        lax.fori_loop(0, NCHUNK, chunk_body, 0)
        pltpu.sync_copy(hcnt, cnt_hbm.at[wid])
        pltpu.sync_copy(hsum, sum_hbm.at[wid])

    return sc_hist


def _sc_pass(lf, basev, shift):
    cnt, sm = _make_sc_hist(shift)(lf, basev)
    cnt = cnt.reshape(ROWS, 2, NBUCK, 16).sum(axis=(1, 3))
    sm = sm.reshape(ROWS, 2, NBUCK, 16).sum(axis=(1, 3))
    return cnt, sm


def _pick(cnt, sm, need):
    cnt_i = cnt.astype(jnp.int32)
    cumtop = jnp.cumsum(cnt_i[:, ::-1], axis=1)[:, ::-1]
    ok = cumtop >= need[:, None]
    jstar = jnp.max(jnp.where(ok, jnp.arange(NBUCK), -1), axis=1)
    jn = jnp.clip(jstar + 1, 0, NBUCK - 1)
    c_above = jnp.where(jstar + 1 < NBUCK,
                        jnp.take_along_axis(cumtop, jn[:, None], axis=1)[:, 0], 0)
    sumtop = jnp.cumsum(sm[:, ::-1], axis=1)[:, ::-1]
    s_above = jnp.where(jstar + 1 < NBUCK,
                        jnp.take_along_axis(sumtop, jn[:, None], axis=1)[:, 0], 0.0)
    return jstar, c_above, s_above


def kernel(net_output, target_structure, bboxes):
    lo = bboxes[..., 0].astype(jnp.int32)
    hi = bboxes[..., 1].astype(jnp.int32)
    loss = _tc_loss(net_output, target_structure, lo, hi)
    lf = loss.reshape(ROWS * SPATIAL)

    zero_base = jnp.zeros((NTILES, 16), jnp.int32)
    c1, s1 = _sc_pass(lf, zero_base, 20)
    need1 = jnp.full((ROWS,), N_TOP, jnp.int32)
    j1, ca1, sa1 = _pick(c1, s1, need1)
    base = (j1 << 20)

    basev = jnp.broadcast_to(base[jnp.arange(NTILES) // 2, None], (NTILES, 16))
    c2, s2 = _sc_pass(lf, basev, 9)
    need2 = need1 - ca1
    j2, ca2, sa2 = _pick(c2, s2, need2)

    tkey = base + (j2 << 9)
    tval = lax.bitcast_convert_type(tkey, jnp.float32)
    row_sum = sa1 + sa2 + (need2 - ca2).astype(jnp.float32) * tval
    return jnp.sum(row_sum) / jnp.float32(ROWS * N_TOP)


# speedup vs baseline: 78.0847x; 2.6505x over previous
import functools

import jax
import jax.numpy as jnp
from jax import lax
from jax.experimental import pallas as pl
from jax.experimental.pallas import tpu as pltpu
from jax.experimental.pallas import tpu_sc as plsc

_INTERPRET = False

B, C, S = 2, 8, 96
SPATIAL = S * S * S
ROWS = B * C
N_TOP = int(round(SPATIAL * 0.1))
ZSLAB = 8
NZ = S // ZSLAB

NTILES = 32
HALF = SPATIAL // 2
CHUNK = 27648
NCHUNK = HALF // CHUNK
NB1 = 2048
NB2 = 1024 + 2
HB1 = NB1 * 16
HB2 = NB2 * 16


def _loss_body(lo_ref, hi_ref, net_ref, tgt_ref, out_ref):
    b = pl.program_id(0)
    s = pl.program_id(1)
    z0 = s * ZSLAB
    r2 = lax.broadcasted_iota(jnp.int32, (S, S), 0)
    c2 = lax.broadcasted_iota(jnp.int32, (S, S), 1)
    dmax = jnp.zeros((ZSLAB, S, S), jnp.float32)

    def bce(x, d):
        return jnp.maximum(x, 0.0) - x * d + jnp.log1p(jnp.exp(-jnp.abs(x)))

    for c in range(C - 1):
        lo0 = lo_ref[b, c, 0]
        lo1 = lo_ref[b, c, 1]
        lo2 = lo_ref[b, c, 2]
        hi0 = hi_ref[b, c, 0]
        hi1 = hi_ref[b, c, 1]
        hi2 = hi_ref[b, c, 2]
        rows = []
        for z in range(ZSLAB):
            zz = z0 + z - lo0
            row = tgt_ref[0, pl.ds(jnp.clip(zz, 0, S - 1), 1), :, :]
            zin = (zz >= 0) & (z0 + z < hi0)
            rows.append(jnp.where(zin, row, 0.0))
        a = jnp.concatenate(rows, axis=0)
        py = ((r2 == (c2 - lo1 + S) % S) & (c2 >= lo1) & (c2 < hi1)
              ).astype(jnp.float32)
        px = ((r2 == (c2 - lo2 + S) % S) & (c2 >= lo2) & (c2 < hi2)
              ).astype(jnp.float32)
        e = lax.dot_general(a, py, (((1,), (0,)), ((), ())),
                            preferred_element_type=jnp.float32)
        d = lax.dot_general(e, px, (((1,), (0,)), ((), ())),
                            preferred_element_type=jnp.float32)
        dmax = jnp.maximum(dmax, d)
        x = net_ref[0, c]
        out_ref[0, c] = bce(x, d)
    x = net_ref[0, C - 1]
    out_ref[0, C - 1] = bce(x, dmax)


def _tc_loss(net, tgt, lo, hi):
    return pl.pallas_call(
        _loss_body,
        grid=(B, NZ),
        in_specs=[
            pl.BlockSpec(memory_space=pltpu.SMEM),
            pl.BlockSpec(memory_space=pltpu.SMEM),
            pl.BlockSpec((1, C, ZSLAB, S, S), lambda b, s: (b, 0, s, 0, 0)),
            pl.BlockSpec((1, S, S, S), lambda b, s: (b, 0, 0, 0)),
        ],
        out_specs=pl.BlockSpec((1, C, ZSLAB, S, S), lambda b, s: (b, 0, s, 0, 0)),
        out_shape=jax.ShapeDtypeStruct((B, C, S, S, S), jnp.float32),
        interpret=_INTERPRET,
    )(lo, hi, net, tgt)


@functools.lru_cache(maxsize=None)
def _make_sc_hist(shift, nbuck, with_sums, clamp):
    mesh = plsc.VectorSubcoreMesh(core_axis_name="c", subcore_axis_name="s")
    hb = nbuck * 16
    nouts = 2 if with_sums else 1
    out_type = [jax.ShapeDtypeStruct((NTILES, hb), jnp.float32)
                for _ in range(nouts)]
    scratch = [pltpu.VMEM((CHUNK,), jnp.float32),
               pltpu.VMEM((CHUNK,), jnp.float32)]
    scratch += [pltpu.VMEM((hb,), jnp.float32) for _ in range(nouts)]
    scratch += [pltpu.VMEM((16,), jnp.int32),
                pltpu.SemaphoreType.DMA,
                pltpu.SemaphoreType.DMA]

    @functools.partial(
        pl.kernel,
        mesh=mesh,
        out_type=out_type,
        scratch_types=scratch,
        compiler_params=pltpu.CompilerParams(needs_layout_passes=False),
    )
    def sc_hist(loss_hbm, base_hbm, *refs):
        if with_sums:
            cnt_hbm, sum_hbm, buf0, buf1, hcnt, hsum, bvec, sem0, sem1 = refs
        else:
            cnt_hbm, buf0, buf1, hcnt, bvec, sem0, sem1 = refs
            hsum = None
        wid = lax.axis_index("s") * 2 + lax.axis_index("c")
        start = wid * HALF
        zeros = jnp.zeros((16,), jnp.float32)

        @plsc.parallel_loop(0, hb // 16, unroll=4)
        def _(i):
            hcnt[pl.ds(i * 16, 16)] = zeros
            if with_sums:
                hsum[pl.ds(i * 16, 16)] = zeros

        pltpu.sync_copy(base_hbm.at[wid], bvec)
        base = bvec[...]
        lane = lax.iota(jnp.int32, 16)

        def _chunk(ci, buf):
            return loss_hbm.at[pl.ds(start + ci * CHUNK, CHUNK)], buf

        def _process(buf):
            ones = jnp.full((16,), 1.0, jnp.float32)

            @plsc.parallel_loop(0, CHUNK // 16, unroll=4)
            def _(i):
                v = buf[pl.ds(i * 16, 16)]
                k = lax.bitcast_convert_type(v, jnp.int32)
                d = (k - base) >> shift
                if clamp:
                    d = jnp.clip(d, -1, nbuck - 2) + 1
                idx = (d << 4) | lane
                plsc.addupdate_scatter(hcnt, [idx], ones)
                if with_sums:
                    plsc.addupdate_scatter(hsum, [idx], v)

        pltpu.async_copy(*_chunk(0, buf0), sem0)

        def outer(ci, _):
            pltpu.async_copy(*_chunk(ci + 1, buf1), sem1)
            pltpu.make_async_copy(*_chunk(ci, buf0), sem0).wait()
            _process(buf0)

            @pl.when(ci + 2 < NCHUNK)
            def _():
                pltpu.async_copy(*_chunk(ci + 2, buf0), sem0)

            pltpu.make_async_copy(*_chunk(ci + 1, buf1), sem1).wait()
            _process(buf1)
            return 0

        lax.fori_loop(0, NCHUNK // 2, lambda i, c: outer(i * 2, c), 0)
        pltpu.sync_copy(hcnt, cnt_hbm.at[wid])
        if with_sums:
            pltpu.sync_copy(hsum, sum_hbm.at[wid])

    return sc_hist


def _sc_pass(lf, basev, shift, nbuck, with_sums, clamp):
    out = _make_sc_hist(shift, nbuck, with_sums, clamp)(lf, basev)
    out = out if isinstance(out, (tuple, list)) else (out,)
    return [o.reshape(ROWS, 2, nbuck, 16).sum(axis=(1, 3)) for o in out]


def _pick(cnt, sm, need, nb):
    cnt_i = cnt.astype(jnp.int32)
    cumtop = jnp.cumsum(cnt_i[:, ::-1], axis=1)[:, ::-1]
    ok = cumtop >= need[:, None]
    jstar = jnp.max(jnp.where(ok, jnp.arange(nb), -1), axis=1)
    jn = jnp.clip(jstar + 1, 0, nb - 1)
    valid = (jstar + 1 < nb)
    c_above = jnp.where(valid,
                        jnp.take_along_axis(cumtop, jn[:, None], axis=1)[:, 0], 0)
    if sm is None:
        return jstar, c_above, None
    sumtop = jnp.cumsum(sm[:, ::-1], axis=1)[:, ::-1]
    s_above = jnp.where(valid,
                        jnp.take_along_axis(sumtop, jn[:, None], axis=1)[:, 0], 0.0)
    return jstar, c_above, s_above


def kernel(net_output, target_structure, bboxes):
    lo = bboxes[..., 0].astype(jnp.int32)
    hi = bboxes[..., 1].astype(jnp.int32)
    loss = _tc_loss(net_output, target_structure, lo, hi)
    lf = loss.reshape(ROWS * SPATIAL)

    need = jnp.full((ROWS,), N_TOP, jnp.int32)
    zero_base = jnp.zeros((NTILES, 16), jnp.int32)
    (c1,) = _sc_pass(lf, zero_base, 20, NB1, False, False)
    j1, _, _ = _pick(c1, None, need, NB1)
    base = (j1 << 20)

    basev = jnp.broadcast_to(base[jnp.arange(NTILES) // 2, None], (NTILES, 16))
    c2, s2 = _sc_pass(lf, basev, 10, NB2, True, True)
    j2, ca2, sa2 = _pick(c2, s2, need, NB2)

    tkey = base + ((j2 - 1) << 10)
    tval = lax.bitcast_convert_type(tkey, jnp.float32)
    row_sum = sa2 + (need - ca2).astype(jnp.float32) * tval
    return jnp.sum(row_sum) / jnp.float32(ROWS * N_TOP)


# speedup vs baseline: 90.6609x; 1.1611x over previous
import functools

import jax
import jax.numpy as jnp
from jax import lax
from jax.experimental import pallas as pl
from jax.experimental.pallas import tpu as pltpu
from jax.experimental.pallas import tpu_sc as plsc

_INTERPRET = False

B, C, S = 2, 8, 96
SPATIAL = S * S * S
ROWS = B * C
N_TOP = int(round(SPATIAL * 0.1))
ZSLAB = 8
NZ = S // ZSLAB

NTILES = 32
XPAD = 128
SPAT2 = S * S * XPAD
HALF = SPAT2 // 2
CHUNK = 36864
NCHUNK = HALF // CHUNK
NB1 = 2048
NB2 = 1024 + 2
HB1 = NB1 * 16
HB2 = NB2 * 16


def _padx(v):
    return lax.pad(v, jnp.float32(0.0), ((0, 0, 0), (0, 0, 0), (0, XPAD - S, 0)))


def _loss_body(lo_ref, hi_ref, net_ref, tgt_ref, out_ref):
    b = pl.program_id(0)
    s = pl.program_id(1)
    z0 = s * ZSLAB
    r2 = lax.broadcasted_iota(jnp.int32, (S, S), 0)
    c2 = lax.broadcasted_iota(jnp.int32, (S, S), 1)
    dmax = jnp.zeros((ZSLAB, S, S), jnp.float32)

    def bce(x, d):
        return jnp.maximum(x, 0.0) - x * d + jnp.log1p(jnp.exp(-jnp.abs(x)))

    for c in range(C - 1):
        lo0 = lo_ref[b, c, 0]
        lo1 = lo_ref[b, c, 1]
        lo2 = lo_ref[b, c, 2]
        hi0 = hi_ref[b, c, 0]
        hi1 = hi_ref[b, c, 1]
        hi2 = hi_ref[b, c, 2]
        rows = []
        for z in range(ZSLAB):
            zz = z0 + z - lo0
            row = tgt_ref[0, pl.ds(jnp.clip(zz, 0, S - 1), 1), :, :]
            zin = (zz >= 0) & (z0 + z < hi0)
            rows.append(jnp.where(zin, row, 0.0))
        a = jnp.concatenate(rows, axis=0)
        py = ((r2 == (c2 - lo1 + S) % S) & (c2 >= lo1) & (c2 < hi1)
              ).astype(jnp.float32)
        px = ((r2 == (c2 - lo2 + S) % S) & (c2 >= lo2) & (c2 < hi2)
              ).astype(jnp.float32)
        e = lax.dot_general(a, py, (((1,), (0,)), ((), ())),
                            preferred_element_type=jnp.float32)
        d = lax.dot_general(e, px, (((1,), (0,)), ((), ())),
                            preferred_element_type=jnp.float32)
        dmax = jnp.maximum(dmax, d)
        x = net_ref[0, c]
        out_ref[0, c] = _padx(bce(x, d))
    x = net_ref[0, C - 1]
    out_ref[0, C - 1] = _padx(bce(x, dmax))


def _tc_loss(net, tgt, lo, hi):
    return pl.pallas_call(
        _loss_body,
        grid=(B, NZ),
        in_specs=[
            pl.BlockSpec(memory_space=pltpu.SMEM),
            pl.BlockSpec(memory_space=pltpu.SMEM),
            pl.BlockSpec((1, C, ZSLAB, S, S), lambda b, s: (b, 0, s, 0, 0)),
            pl.BlockSpec((1, S, S, S), lambda b, s: (b, 0, 0, 0)),
        ],
        out_specs=pl.BlockSpec((1, C, ZSLAB, S, XPAD),
                               lambda b, s: (b, 0, s, 0, 0)),
        out_shape=jax.ShapeDtypeStruct((B, C, S, S, XPAD), jnp.float32),
        interpret=_INTERPRET,
    )(lo, hi, net, tgt)


@functools.lru_cache(maxsize=None)
def _make_sc_hist(shift, nbuck, with_sums, clamp):
    mesh = plsc.VectorSubcoreMesh(core_axis_name="c", subcore_axis_name="s")
    hb = nbuck * 16
    nouts = 2 if with_sums else 1
    out_type = [jax.ShapeDtypeStruct((NTILES, hb), jnp.float32)
                for _ in range(nouts)]
    scratch = [pltpu.VMEM((CHUNK,), jnp.float32),
               pltpu.VMEM((CHUNK,), jnp.float32)]
    scratch += [pltpu.VMEM((hb,), jnp.float32) for _ in range(nouts)]
    scratch += [pltpu.VMEM((16,), jnp.int32),
                pltpu.SemaphoreType.DMA,
                pltpu.SemaphoreType.DMA]

    @functools.partial(
        pl.kernel,
        mesh=mesh,
        out_type=out_type,
        scratch_types=scratch,
        compiler_params=pltpu.CompilerParams(needs_layout_passes=False),
    )
    def sc_hist(loss_hbm, base_hbm, *refs):
        if with_sums:
            cnt_hbm, sum_hbm, buf0, buf1, hcnt, hsum, bvec, sem0, sem1 = refs
        else:
            cnt_hbm, buf0, buf1, hcnt, bvec, sem0, sem1 = refs
            hsum = None
        wid = lax.axis_index("s") * 2 + lax.axis_index("c")
        start = wid * HALF
        zeros = jnp.zeros((16,), jnp.float32)

        @plsc.parallel_loop(0, hb // 16, unroll=4)
        def _(i):
            hcnt[pl.ds(i * 16, 16)] = zeros
            if with_sums:
                hsum[pl.ds(i * 16, 16)] = zeros

        pltpu.sync_copy(base_hbm.at[wid], bvec)
        base = bvec[...]
        lane = lax.iota(jnp.int32, 16)

        def _chunk(ci, buf):
            return loss_hbm.at[pl.ds(start + ci * CHUNK, CHUNK)], buf

        def _process(buf):
            ones = jnp.full((16,), 1.0, jnp.float32)

            @plsc.parallel_loop(0, CHUNK // 16, unroll=4)
            def _(i):
                v = buf[pl.ds(i * 16, 16)]
                k = lax.bitcast_convert_type(v, jnp.int32)
                d = (k - base) >> shift
                if clamp:
                    d = jnp.clip(d, -1, nbuck - 2) + 1
                idx = (d << 4) | lane
                plsc.addupdate_scatter(hcnt, [idx], ones)
                if with_sums:
                    plsc.addupdate_scatter(hsum, [idx], v)

        pltpu.async_copy(*_chunk(0, buf0), sem0)

        def outer(ci, _):
            pltpu.async_copy(*_chunk(ci + 1, buf1), sem1)
            pltpu.make_async_copy(*_chunk(ci, buf0), sem0).wait()
            _process(buf0)

            @pl.when(ci + 2 < NCHUNK)
            def _():
                pltpu.async_copy(*_chunk(ci + 2, buf0), sem0)

            pltpu.make_async_copy(*_chunk(ci + 1, buf1), sem1).wait()
            _process(buf1)
            return 0

        lax.fori_loop(0, NCHUNK // 2, lambda i, c: outer(i * 2, c), 0)
        pltpu.sync_copy(hcnt, cnt_hbm.at[wid])
        if with_sums:
            pltpu.sync_copy(hsum, sum_hbm.at[wid])

    return sc_hist


def _sc_pass(lf, basev, shift, nbuck, with_sums, clamp):
    out = _make_sc_hist(shift, nbuck, with_sums, clamp)(lf, basev)
    out = out if isinstance(out, (tuple, list)) else (out,)
    return [o.reshape(ROWS, 2, nbuck, 16).sum(axis=(1, 3)) for o in out]


def _pick(cnt, sm, need, nb):
    cnt_i = cnt.astype(jnp.int32)
    cumtop = jnp.cumsum(cnt_i[:, ::-1], axis=1)[:, ::-1]
    ok = cumtop >= need[:, None]
    jstar = jnp.max(jnp.where(ok, jnp.arange(nb), -1), axis=1)
    jn = jnp.clip(jstar + 1, 0, nb - 1)
    valid = (jstar + 1 < nb)
    c_above = jnp.where(valid,
                        jnp.take_along_axis(cumtop, jn[:, None], axis=1)[:, 0], 0)
    if sm is None:
        return jstar, c_above, None
    sumtop = jnp.cumsum(sm[:, ::-1], axis=1)[:, ::-1]
    s_above = jnp.where(valid,
                        jnp.take_along_axis(sumtop, jn[:, None], axis=1)[:, 0], 0.0)
    return jstar, c_above, s_above


def kernel(net_output, target_structure, bboxes):
    lo = bboxes[..., 0].astype(jnp.int32)
    hi = bboxes[..., 1].astype(jnp.int32)
    loss = _tc_loss(net_output, target_structure, lo, hi)
    lf = loss.reshape(ROWS * SPAT2)

    need = jnp.full((ROWS,), N_TOP, jnp.int32)
    zero_base = jnp.zeros((NTILES, 16), jnp.int32)
    (c1,) = _sc_pass(lf, zero_base, 20, NB1, False, False)
    j1, _, _ = _pick(c1, None, need, NB1)
    base = (j1 << 20)

    basev = jnp.broadcast_to(base[jnp.arange(NTILES) // 2, None], (NTILES, 16))
    c2, s2 = _sc_pass(lf, basev, 10, NB2, True, True)
    j2, ca2, sa2 = _pick(c2, s2, need, NB2)

    tkey = base + ((j2 - 1) << 10)
    tval = lax.bitcast_convert_type(tkey, jnp.float32)
    row_sum = sa2 + (need - ca2).astype(jnp.float32) * tval
    return jnp.sum(row_sum) / jnp.float32(ROWS * N_TOP)


# speedup vs baseline: 101.5425x; 1.1200x over previous
import functools

import jax
import jax.numpy as jnp
from jax import lax
from jax.experimental import pallas as pl
from jax.experimental.pallas import tpu as pltpu
from jax.experimental.pallas import tpu_sc as plsc

_INTERPRET = False

B, C, S = 2, 8, 96
SPATIAL = S * S * S
ROWS = B * C
N_TOP = int(round(SPATIAL * 0.1))
ZSLAB = 8
NZ = S // ZSLAB

NTILES = 32
XPAD = 128
SPAT2 = S * S * XPAD
HALF = SPAT2 // 2
NRCH = 288
NROWS_TILE = (ROWS * S * S) // NTILES
NCHUNK = NROWS_TILE // NRCH
CHUNK = NRCH * S
NB1 = 2048
NB2 = 1024 + 2
HB1 = NB1 * 16
HB2 = NB2 * 16


def _padx(v):
    return lax.pad(v, jnp.float32(0.0), ((0, 0, 0), (0, 0, 0), (0, XPAD - S, 0)))


def _loss_body(lo_ref, hi_ref, net_ref, tgt_ref, out_ref):
    b = pl.program_id(0)
    s = pl.program_id(1)
    z0 = s * ZSLAB
    r2 = lax.broadcasted_iota(jnp.int32, (S, S), 0)
    c2 = lax.broadcasted_iota(jnp.int32, (S, S), 1)
    dmax = jnp.zeros((ZSLAB, S, S), jnp.float32)

    def bce(x, d):
        return jnp.maximum(x, 0.0) - x * d + jnp.log1p(jnp.exp(-jnp.abs(x)))

    for c in range(C - 1):
        lo0 = lo_ref[b, c, 0]
        lo1 = lo_ref[b, c, 1]
        lo2 = lo_ref[b, c, 2]
        hi0 = hi_ref[b, c, 0]
        hi1 = hi_ref[b, c, 1]
        hi2 = hi_ref[b, c, 2]
        rows = []
        for z in range(ZSLAB):
            zz = z0 + z - lo0
            row = tgt_ref[0, pl.ds(jnp.clip(zz, 0, S - 1), 1), :, :]
            zin = (zz >= 0) & (z0 + z < hi0)
            rows.append(jnp.where(zin, row, 0.0))
        a = jnp.concatenate(rows, axis=0)
        py = ((r2 == (c2 - lo1 + S) % S) & (c2 >= lo1) & (c2 < hi1)
              ).astype(jnp.float32)
        px = ((r2 == (c2 - lo2 + S) % S) & (c2 >= lo2) & (c2 < hi2)
              ).astype(jnp.float32)
        e = lax.dot_general(a, py, (((1,), (0,)), ((), ())),
                            preferred_element_type=jnp.float32)
        d = lax.dot_general(e, px, (((1,), (0,)), ((), ())),
                            preferred_element_type=jnp.float32)
        dmax = jnp.maximum(dmax, d)
        x = net_ref[0, c]
        out_ref[0, c] = _padx(bce(x, d))
    x = net_ref[0, C - 1]
    out_ref[0, C - 1] = _padx(bce(x, dmax))


def _tc_loss(net, tgt, lo, hi):
    return pl.pallas_call(
        _loss_body,
        grid=(B, NZ),
        in_specs=[
            pl.BlockSpec(memory_space=pltpu.SMEM),
            pl.BlockSpec(memory_space=pltpu.SMEM),
            pl.BlockSpec((1, C, ZSLAB, S, S), lambda b, s: (b, 0, s, 0, 0)),
            pl.BlockSpec((1, S, S, S), lambda b, s: (b, 0, 0, 0)),
        ],
        out_specs=pl.BlockSpec((1, C, ZSLAB, S, XPAD),
                               lambda b, s: (b, 0, s, 0, 0)),
        out_shape=jax.ShapeDtypeStruct((B, C, S, S, XPAD), jnp.float32),
        interpret=_INTERPRET,
    )(lo, hi, net, tgt)


@functools.lru_cache(maxsize=None)
def _make_sc_hist(shift, nbuck, with_sums, clamp):
    mesh = plsc.VectorSubcoreMesh(core_axis_name="c", subcore_axis_name="s")
    hb = nbuck * 16
    nouts = 2 if with_sums else 1
    out_type = [jax.ShapeDtypeStruct((NTILES, hb), jnp.float32)
                for _ in range(nouts)]
    scratch = [pltpu.VMEM((NRCH, S), jnp.float32),
               pltpu.VMEM((NRCH, S), jnp.float32)]
    scratch += [pltpu.VMEM((hb,), jnp.float32) for _ in range(nouts)]
    scratch += [pltpu.VMEM((16,), jnp.int32),
                pltpu.SemaphoreType.DMA,
                pltpu.SemaphoreType.DMA]

    @functools.partial(
        pl.kernel,
        mesh=mesh,
        out_type=out_type,
        scratch_types=scratch,
        compiler_params=pltpu.CompilerParams(needs_layout_passes=False,
                                             use_tc_tiling_on_sc=False),
    )
    def sc_hist(loss_hbm, base_hbm, *refs):
        if with_sums:
            cnt_hbm, sum_hbm, buf0, buf1, hcnt, hsum, bvec, sem0, sem1 = refs
        else:
            cnt_hbm, buf0, buf1, hcnt, bvec, sem0, sem1 = refs
            hsum = None
        wid = lax.axis_index("s") * 2 + lax.axis_index("c")
        start = wid * NROWS_TILE
        zeros = jnp.zeros((16,), jnp.float32)

        @plsc.parallel_loop(0, hb // 16, unroll=4)
        def _(i):
            hcnt[pl.ds(i * 16, 16)] = zeros
            if with_sums:
                hsum[pl.ds(i * 16, 16)] = zeros

        pltpu.sync_copy(base_hbm.at[wid], bvec)
        base = bvec[...]
        lane = lax.iota(jnp.int32, 16)

        def _chunk(ci, buf):
            return loss_hbm.at[pl.ds(start + ci * NRCH, NRCH), pl.ds(0, S)], buf

        def _process(buf):
            ones = jnp.full((16,), 1.0, jnp.float32)

            @plsc.parallel_loop(0, NRCH, unroll=2)
            def _(r):
                for j in range(S // 16):
                    v = buf[r, pl.ds(j * 16, 16)]
                    k = lax.bitcast_convert_type(v, jnp.int32)
                    d = (k - base) >> shift
                    if clamp:
                        d = jnp.clip(d, -1, nbuck - 2) + 1
                    idx = (d << 4) | lane
                    plsc.addupdate_scatter(hcnt, [idx], ones)
                    if with_sums:
                        plsc.addupdate_scatter(hsum, [idx], v)

        pltpu.async_copy(*_chunk(0, buf0), sem0)

        def outer(ci, _):
            pltpu.async_copy(*_chunk(ci + 1, buf1), sem1)
            pltpu.make_async_copy(*_chunk(ci, buf0), sem0).wait()
            _process(buf0)

            @pl.when(ci + 2 < NCHUNK)
            def _():
                pltpu.async_copy(*_chunk(ci + 2, buf0), sem0)

            pltpu.make_async_copy(*_chunk(ci + 1, buf1), sem1).wait()
            _process(buf1)
            return 0

        lax.fori_loop(0, NCHUNK // 2, lambda i, c: outer(i * 2, c), 0)
        pltpu.sync_copy(hcnt, cnt_hbm.at[wid])
        if with_sums:
            pltpu.sync_copy(hsum, sum_hbm.at[wid])

    return sc_hist


def _sc_pass(lf, basev, shift, nbuck, with_sums, clamp):
    out = _make_sc_hist(shift, nbuck, with_sums, clamp)(lf, basev)
    out = out if isinstance(out, (tuple, list)) else (out,)
    return [o.reshape(ROWS, 2, nbuck, 16).sum(axis=(1, 3)) for o in out]


def _pick(cnt, sm, need, nb):
    cnt_i = cnt.astype(jnp.int32)
    cumtop = jnp.cumsum(cnt_i[:, ::-1], axis=1)[:, ::-1]
    ok = cumtop >= need[:, None]
    jstar = jnp.max(jnp.where(ok, jnp.arange(nb), -1), axis=1)
    jn = jnp.clip(jstar + 1, 0, nb - 1)
    valid = (jstar + 1 < nb)
    c_above = jnp.where(valid,
                        jnp.take_along_axis(cumtop, jn[:, None], axis=1)[:, 0], 0)
    if sm is None:
        return jstar, c_above, None
    sumtop = jnp.cumsum(sm[:, ::-1], axis=1)[:, ::-1]
    s_above = jnp.where(valid,
                        jnp.take_along_axis(sumtop, jn[:, None], axis=1)[:, 0], 0.0)
    return jstar, c_above, s_above


def kernel(net_output, target_structure, bboxes):
    lo = bboxes[..., 0].astype(jnp.int32)
    hi = bboxes[..., 1].astype(jnp.int32)
    loss = _tc_loss(net_output, target_structure, lo, hi)
    lf = loss.reshape(ROWS * S * S, XPAD)

    need = jnp.full((ROWS,), N_TOP, jnp.int32)
    zero_base = jnp.zeros((NTILES, 16), jnp.int32)
    (c1,) = _sc_pass(lf, zero_base, 20, NB1, False, False)
    j1, _, _ = _pick(c1, None, need, NB1)
    base = (j1 << 20)

    basev = jnp.broadcast_to(base[jnp.arange(NTILES) // 2, None], (NTILES, 16))
    c2, s2 = _sc_pass(lf, basev, 10, NB2, True, True)
    j2, ca2, sa2 = _pick(c2, s2, need, NB2)

    tkey = base + ((j2 - 1) << 10)
    tval = lax.bitcast_convert_type(tkey, jnp.float32)
    row_sum = sa2 + (need - ca2).astype(jnp.float32) * tval
    return jnp.sum(row_sum) / jnp.float32(ROWS * N_TOP)
